# Initial kernel scaffold; baseline (speedup 1.0000x reference)
import jax
import jax.numpy as jnp
from jax.experimental import pallas as pl


def kernel(p_feats, batch_ids, gamma, betas):
    raise NotImplementedError("write your pallas kernel here")


# speedup vs baseline: 81.5716x; 81.5716x over previous
import functools

import jax
import jax.numpy as jnp
from jax.experimental import pallas as pl
from jax.experimental.pallas import tpu as pltpu

NF = 128
NG = 16
GS = NF // NG
NS = 16
EPS = 1e-8
BLK = 3200


def _stats_body(x_ref, ids_ref, s_ref, q_ref, c_ref):
    pid = pl.program_id(0)

    @pl.when(pid == 0)
    def _init():
        s_ref[...] = jnp.zeros_like(s_ref)
        q_ref[...] = jnp.zeros_like(q_ref)
        c_ref[...] = jnp.zeros_like(c_ref)

    x = x_ref[...]
    ids = ids_ref[0]
    seg_iota = jax.lax.broadcasted_iota(jnp.int32, (NS, BLK), 0)
    onehot_t = (ids == seg_iota).astype(jnp.float32)
    dot = functools.partial(
        jax.lax.dot_general,
        dimension_numbers=(((1,), (0,)), ((), ())),
        precision=jax.lax.Precision.HIGHEST,
        preferred_element_type=jnp.float32,
    )
    s_ref[...] += dot(onehot_t, x)
    q_ref[...] += dot(onehot_t, x * x)
    c_ref[...] += jnp.broadcast_to(
        jnp.sum(onehot_t, axis=1, keepdims=True), (NS, NF))


def _norm_body(s_ref, q_ref, c_ref, g_ref, b_ref, x_ref, ids_ref, o_ref):
    fg_iota_f = jax.lax.broadcasted_iota(jnp.int32, (NF, NG), 0)
    fg_iota_g = jax.lax.broadcasted_iota(jnp.int32, (NF, NG), 1)
    gmat = (fg_iota_f % NG == fg_iota_g).astype(jnp.float32)

    dot_ff = functools.partial(
        jax.lax.dot_general,
        dimension_numbers=(((1,), (0,)), ((), ())),
        precision=jax.lax.Precision.HIGHEST,
        preferred_element_type=jnp.float32,
    )
    sg = dot_ff(s_ref[...], gmat)
    qg = dot_ff(q_ref[...], gmat)
    cn = c_ref[...][:, :NG]
    n = jnp.maximum(cn * GS, 1.0)
    mean = sg / n
    var = qg / n - mean * mean
    rstd = jax.lax.rsqrt(var + EPS)
    gmat_t = (fg_iota_g.T == fg_iota_f.T % NG).astype(jnp.float32)
    meanf = dot_ff(mean, gmat_t)
    rstdf = dot_ff(rstd, gmat_t)
    scale = rstdf * g_ref[...]
    bias = b_ref[...] - meanf * scale

    ids = ids_ref[0]
    seg_iota = jax.lax.broadcasted_iota(jnp.int32, (NS, BLK), 0)
    onehot_t = (ids == seg_iota).astype(jnp.float32)
    dot_sel = functools.partial(
        jax.lax.dot_general,
        dimension_numbers=(((0,), (0,)), ((), ())),
        precision=jax.lax.Precision.HIGHEST,
        preferred_element_type=jnp.float32,
    )
    srow = dot_sel(onehot_t, scale)
    brow = dot_sel(onehot_t, bias)
    o_ref[...] = x_ref[...] * srow + brow


def kernel(p_feats, batch_ids, gamma, betas):
    n_rows = p_feats.shape[0]
    nblk = n_rows // BLK
    ids3 = batch_ids.astype(jnp.int32).reshape(nblk, 1, BLK)

    stats = pl.pallas_call(
        _stats_body,
        grid=(nblk,),
        in_specs=[
            pl.BlockSpec((BLK, NF), lambda i: (i, 0)),
            pl.BlockSpec((1, 1, BLK), lambda i: (i, 0, 0)),
        ],
        out_specs=[
            pl.BlockSpec((NS, NF), lambda i: (0, 0)),
            pl.BlockSpec((NS, NF), lambda i: (0, 0)),
            pl.BlockSpec((NS, NF), lambda i: (0, 0)),
        ],
        out_shape=[
            jax.ShapeDtypeStruct((NS, NF), jnp.float32),
            jax.ShapeDtypeStruct((NS, NF), jnp.float32),
            jax.ShapeDtypeStruct((NS, NF), jnp.float32),
        ],
    )
    s, q, c = stats(p_feats, ids3)

    out = pl.pallas_call(
        _norm_body,
        grid=(nblk,),
        in_specs=[
            pl.BlockSpec((NS, NF), lambda i: (0, 0)),
            pl.BlockSpec((NS, NF), lambda i: (0, 0)),
            pl.BlockSpec((NS, NF), lambda i: (0, 0)),
            pl.BlockSpec((1, NF), lambda i: (0, 0)),
            pl.BlockSpec((1, NF), lambda i: (0, 0)),
            pl.BlockSpec((BLK, NF), lambda i: (i, 0)),
            pl.BlockSpec((1, 1, BLK), lambda i: (i, 0, 0)),
        ],
        out_specs=pl.BlockSpec((BLK, NF), lambda i: (i, 0)),
        out_shape=jax.ShapeDtypeStruct((n_rows, NF), jnp.float32),
    )(s, q, c, gamma, betas, p_feats, ids3)
    return out
